# baseline (speedup 1.0000x reference)
import functools

import jax
import jax.numpy as jnp
from jax import lax
from jax.experimental import pallas as pl
from jax.experimental.pallas import tpu as pltpu
from jax.experimental.pallas import tpu_sc as plsc

F32 = jnp.float32
I32 = jnp.int32

NC = 2
NS = 16
NW = NC * NS
L = 16
CHUNK = 4096


def _sin_poly(x):
    z = x * x
    p = (-1.9515295891e-4 * z + 8.3321608736e-3) * z - 1.6666654611e-1
    return x + x * z * p


def _cos_poly(x):
    z = x * x
    p = (2.443315711809948e-5 * z - 1.388731625493765e-3) * z + 4.166664568298827e-2
    return 1.0 + z * (p * z - 0.5)


def _rsqrt_nr(x, iters):
    i = lax.bitcast_convert_type(x, I32)
    i = jnp.int32(0x5F3759DF) - lax.shift_right_arithmetic(i, 1)
    y = lax.bitcast_convert_type(i, F32)
    for _ in range(iters):
        y = y * (1.5 - 0.5 * x * y * y)
    return y


def _sqrt_sc(x, iters=3):
    return x * _rsqrt_nr(x, iters)


def _atan2_sc(y, x):
    ax = jnp.abs(x)
    ay = jnp.abs(y)
    hi = jnp.maximum(ax, ay)
    lo = jnp.minimum(ax, ay)
    t = lo / (hi + 1e-37)
    u = t * t
    p = jnp.float32(-0.0117212)
    p = p * u + 0.05265332
    p = p * u - 0.11643287
    p = p * u + 0.19354346
    p = p * u - 0.33262347
    p = p * u + 0.99997726
    a = t * p
    a = jnp.where(ay > ax, 1.5707963267948966 - a, a)
    a = jnp.where(x < 0.0, 3.141592653589793 - a, a)
    return jnp.where(y < 0.0, -a, a)


def _edge_math(r, th, ph, sp, tp, bx, by, wx, wy, sxv, syv):
    cth = _cos_poly(th)
    sth = _sin_poly(th)
    zp = ph * ph
    cph = 1.0 + zp * (4.1666668e-2 * zp - 0.5)
    sph = ph + ph * zp * (8.3333338e-3 * zp - 1.6666667e-1)
    rcp = r * cph
    vx = rcp * cth
    vy = rcp * sth
    vz = r * sph
    ux = sp[0] * vx + sp[1] * vy + sp[2] * vz + (sp[9] - tp[9])
    uy = sp[3] * vx + sp[4] * vy + sp[5] * vz + (sp[10] - tp[10])
    uz = sp[6] * vx + sp[7] * vy + sp[8] * vz + (sp[11] - tp[11])
    lx = tp[0] * ux + tp[3] * uy + tp[6] * uz
    ly = tp[1] * ux + tp[4] * uy + tp[7] * uz
    lz = tp[2] * ux + tp[5] * uy + tp[8] * uz
    rr = _sqrt_sc(lx * lx + ly * ly + lz * lz + 1e-12)
    tt = _atan2_sc(ly, lx)
    ex = (rr * sxv - bx) * _sqrt_sc(wx + 1e-8, 2)
    ey = (tt * syv - by) * _sqrt_sc(wy + 1e-8, 2)
    return ex, ey


def _sc_call(n_edges, n_poses):
    pw = n_edges // NW
    n_chunks = pw // CHUNK
    groups = CHUNK // L
    mesh = plsc.VectorSubcoreMesh(core_axis_name="c", subcore_axis_name="s")

    @functools.partial(
        pl.kernel,
        out_type=jax.ShapeDtypeStruct((2 * n_edges,), F32),
        mesh=mesh,
        compiler_params=pltpu.CompilerParams(
            needs_layout_passes=False, use_tc_tiling_on_sc=False),
        scratch_types=[
            pltpu.VMEM((n_poses * 8,), F32),
            pltpu.VMEM((n_poses * 17,), F32),
            pltpu.VMEM((L,), F32),
            pltpu.VMEM((L,), F32),
        ] + 2 * [
            pltpu.VMEM((CHUNK // 128, 2, 128), F32),
            pltpu.VMEM((CHUNK,), F32),
            pltpu.VMEM((CHUNK // 128, 2, 128), F32),
            pltpu.VMEM((2 * CHUNK,), F32),
            pltpu.VMEM((CHUNK,), I32),
            pltpu.VMEM((CHUNK,), I32),
            pltpu.VMEM((2 * CHUNK,), F32),
        ] + [
            pltpu.SemaphoreType.DMA,
            pltpu.SemaphoreType.DMA,
            pltpu.SemaphoreType.DMA,
            pltpu.SemaphoreType.DMA,
        ],
    )
    def run(pose_hbm, rth_hbm, elev_hbm, base_hbm, w_hbm, sidx_hbm, tidx_hbm,
            sx_hbm, sy_hbm, out_hbm,
            pose_v, rt_v, sx_v, sy_v,
            rth_v0, elev_v0, base_v0, w_v0, sidx_v0, tidx_v0, out_v0,
            rth_v1, elev_v1, base_v1, w_v1, sidx_v1, tidx_v1, out_v1,
            sem_in0, sem_in1, sem_out0, sem_out1):
        wid = lax.axis_index("s") * NC + lax.axis_index("c")
        base0 = wid * pw
        nblk = CHUNK // 128
        pltpu.sync_copy(pose_hbm, pose_v)
        pltpu.sync_copy(sx_hbm, sx_v)
        pltpu.sync_copy(sy_hbm, sy_v)
        sxv = sx_v[...]
        syv = sy_v[...]
        iota = lax.iota(I32, L)
        iota2 = 2 * iota

        def build_rt(g, carry):
            pidx = g * L + iota
            a = lax.shift_left(pidx, 3)
            tx0 = plsc.load_gather(pose_v, [a])
            ty0 = plsc.load_gather(pose_v, [a + 1])
            tz0 = plsc.load_gather(pose_v, [a + 2])
            qx = plsc.load_gather(pose_v, [a + 3])
            qy = plsc.load_gather(pose_v, [a + 4])
            qz = plsc.load_gather(pose_v, [a + 5])
            qw = plsc.load_gather(pose_v, [a + 6])
            x2 = qx + qx
            y2 = qy + qy
            z2 = qz + qz
            xx2 = x2 * qx
            yy2 = y2 * qy
            zz2 = z2 * qz
            xy2 = x2 * qy
            xz2 = x2 * qz
            yz2 = y2 * qz
            wx2 = x2 * qw
            wy2 = y2 * qw
            wz2 = z2 * qw
            d = lax.shift_left(pidx, 4) + pidx
            plsc.store_scatter(rt_v, [d], 1.0 - (yy2 + zz2))
            plsc.store_scatter(rt_v, [d + 1], xy2 - wz2)
            plsc.store_scatter(rt_v, [d + 2], xz2 + wy2)
            plsc.store_scatter(rt_v, [d + 3], xy2 + wz2)
            plsc.store_scatter(rt_v, [d + 4], 1.0 - (xx2 + zz2))
            plsc.store_scatter(rt_v, [d + 5], yz2 - wx2)
            plsc.store_scatter(rt_v, [d + 6], xz2 - wy2)
            plsc.store_scatter(rt_v, [d + 7], yz2 + wx2)
            plsc.store_scatter(rt_v, [d + 8], 1.0 - (xx2 + yy2))
            plsc.store_scatter(rt_v, [d + 9], tx0)
            plsc.store_scatter(rt_v, [d + 10], ty0)
            plsc.store_scatter(rt_v, [d + 11], tz0)
            return carry

        lax.fori_loop(0, n_poses // L, build_rt, 0)

        slots = [
            (rth_v0, elev_v0, base_v0, w_v0, sidx_v0, tidx_v0, out_v0,
             sem_in0, sem_out0),
            (rth_v1, elev_v1, base_v1, w_v1, sidx_v1, tidx_v1, out_v1,
             sem_in1, sem_out1),
        ]

        def issue_in(c):
            rth_v, elev_v, base_v, w_v, sidx_v, tidx_v, _, sem, _ = slots[c % 2]
            base = base0 + c * CHUNK
            blk0 = base // 128
            return [
                pltpu.async_copy(rth_hbm.at[pl.ds(blk0, nblk)], rth_v, sem),
                pltpu.async_copy(elev_hbm.at[pl.ds(base, CHUNK)], elev_v, sem),
                pltpu.async_copy(base_hbm.at[pl.ds(blk0, nblk)], base_v, sem),
                pltpu.async_copy(w_hbm.at[pl.ds(2 * base, 2 * CHUNK)], w_v, sem),
                pltpu.async_copy(sidx_hbm.at[pl.ds(base, CHUNK)], sidx_v, sem),
                pltpu.async_copy(tidx_hbm.at[pl.ds(base, CHUNK)], tidx_v, sem),
            ]

        def compute_chunk(c):
            rth_v, elev_v, base_v, w_v, sidx_v, tidx_v, out_v, _, _ = slots[c % 2]

            def grp(j, carry2):
                off = j * L
                b = lax.shift_right_logical(j, 3)
                o = lax.shift_left(lax.bitwise_and(j, 7), 4)
                si = sidx_v[pl.ds(off, L)]
                ti = tidx_v[pl.ds(off, L)]
                sa = lax.shift_left(si, 4) + si
                ta = lax.shift_left(ti, 4) + ti
                sp = [plsc.load_gather(rt_v, [sa + k]) for k in range(12)]
                tp = [plsc.load_gather(rt_v, [ta + k]) for k in range(12)]
                r = rth_v[b, 0, pl.ds(o, L)]
                th = rth_v[b, 1, pl.ds(o, L)]
                ph = elev_v[pl.ds(off, L)]
                bx = base_v[b, 0, pl.ds(o, L)]
                by = base_v[b, 1, pl.ds(o, L)]
                i2 = 2 * off + iota2
                wx = plsc.load_gather(w_v, [i2])
                wy = plsc.load_gather(w_v, [i2 + 1])
                ex, ey = _edge_math(r, th, ph, sp, tp, bx, by, wx, wy, sxv, syv)
                plsc.store_scatter(out_v, [i2], ex)
                plsc.store_scatter(out_v, [i2 + 1], ey)
                return carry2

            lax.fori_loop(0, groups, grp, 0, unroll=2)

        def issue_out(c):
            out_v = slots[c % 2][6]
            sem = slots[c % 2][8]
            base = base0 + c * CHUNK
            return pltpu.async_copy(out_v, out_hbm.at[pl.ds(2 * base, 2 * CHUNK)],
                                    sem)

        pending_in = {0: issue_in(0)}
        pending_out = {}
        for c in range(n_chunks):
            for d in pending_in.pop(c):
                d.wait()
            if c + 1 < n_chunks:
                pending_in[c + 1] = issue_in(c + 1)
            if c - 2 in pending_out:
                pending_out.pop(c - 2).wait()
            compute_chunk(c)
            pending_out[c] = issue_out(c)
        for c in sorted(pending_out):
            pending_out.pop(c).wait()

    return run


def kernel(poses, elevation_angle_active, patch_coords_r_theta, coords_baseline,
           weights_1d, scale, source_frame_idx, target_frame_idx, patch_idx,
           inverse_patch_idx):
    n_edges = source_frame_idx.shape[0]
    n_poses = poses.shape[1]
    pose_pad = jnp.concatenate(
        [poses[0], jnp.zeros((n_poses, 1), F32)], axis=-1).reshape(-1)
    sx = jnp.broadcast_to(scale.reshape(2)[0], (L,))
    sy = jnp.broadcast_to(scale.reshape(2)[1], (L,))
    nb = n_edges // 128
    rth3 = patch_coords_r_theta.reshape(nb, 128, 2).transpose(0, 2, 1)
    base3 = coords_baseline.reshape(nb, 128, 2).transpose(0, 2, 1)
    elev1 = elevation_angle_active.reshape(n_edges)
    run = _sc_call(n_edges, n_poses)
    out = run(pose_pad, rth3, elev1, base3, weights_1d,
              source_frame_idx, target_frame_idx, sx, sy)
    tcw = jnp.sum(jnp.sin(weights_1d) * jnp.cos(weights_1d))
    return out.reshape(1, 2 * n_edges) + 0.0 * tcw


# speedup vs baseline: 1.0406x; 1.0406x over previous
import functools

import jax
import jax.numpy as jnp
from jax import lax
from jax.experimental import pallas as pl
from jax.experimental.pallas import tpu as pltpu
from jax.experimental.pallas import tpu_sc as plsc

F32 = jnp.float32
I32 = jnp.int32

NC = 2
NS = 16
NW = NC * NS
L = 16
CHUNK = 4096


def _sin_poly(x):
    z = x * x
    p = (-1.9515295891e-4 * z + 8.3321608736e-3) * z - 1.6666654611e-1
    return x + x * z * p


def _cos_poly(x):
    z = x * x
    p = (2.443315711809948e-5 * z - 1.388731625493765e-3) * z + 4.166664568298827e-2
    return 1.0 + z * (p * z - 0.5)


def _rsqrt_nr(x, iters):
    i = lax.bitcast_convert_type(x, I32)
    i = jnp.int32(0x5F3759DF) - lax.shift_right_arithmetic(i, 1)
    y = lax.bitcast_convert_type(i, F32)
    for _ in range(iters):
        y = y * (1.5 - 0.5 * x * y * y)
    return y


def _sqrt_sc(x, iters=3):
    return x * _rsqrt_nr(x, iters)


def _atan2_sc(y, x):
    ax = jnp.abs(x)
    ay = jnp.abs(y)
    hi = jnp.maximum(ax, ay)
    lo = jnp.minimum(ax, ay)
    t = lo / (hi + 1e-37)
    u = t * t
    p = jnp.float32(-0.0117212)
    p = p * u + 0.05265332
    p = p * u - 0.11643287
    p = p * u + 0.19354346
    p = p * u - 0.33262347
    p = p * u + 0.99997726
    a = t * p
    a = jnp.where(ay > ax, 1.5707963267948966 - a, a)
    a = jnp.where(x < 0.0, 3.141592653589793 - a, a)
    return jnp.where(y < 0.0, -a, a)


def _edge_math(r, th, ph, sp, tp, bx, by, wx, wy, sxv, syv):
    cth = _cos_poly(th)
    sth = _sin_poly(th)
    zp = ph * ph
    cph = 1.0 + zp * (4.1666668e-2 * zp - 0.5)
    sph = ph + ph * zp * (8.3333338e-3 * zp - 1.6666667e-1)
    rcp = r * cph
    vx = rcp * cth
    vy = rcp * sth
    vz = r * sph
    ux = sp[0] * vx + sp[1] * vy + sp[2] * vz + (sp[9] - tp[9])
    uy = sp[3] * vx + sp[4] * vy + sp[5] * vz + (sp[10] - tp[10])
    uz = sp[6] * vx + sp[7] * vy + sp[8] * vz + (sp[11] - tp[11])
    lx = tp[0] * ux + tp[3] * uy + tp[6] * uz
    ly = tp[1] * ux + tp[4] * uy + tp[7] * uz
    lz = tp[2] * ux + tp[5] * uy + tp[8] * uz
    rr = _sqrt_sc(lx * lx + ly * ly + lz * lz + 1e-12)
    tt = _atan2_sc(ly, lx)
    ex = (rr * sxv - bx) * _sqrt_sc(wx + 1e-8, 2)
    ey = (tt * syv - by) * _sqrt_sc(wy + 1e-8, 2)
    return ex, ey


def _sc_call(n_edges, n_poses):
    pw = n_edges // NW
    n_chunks = pw // CHUNK
    groups = CHUNK // L
    mesh = plsc.VectorSubcoreMesh(core_axis_name="c", subcore_axis_name="s")

    @functools.partial(
        pl.kernel,
        out_type=jax.ShapeDtypeStruct((2 * n_edges,), F32),
        mesh=mesh,
        compiler_params=pltpu.CompilerParams(
            needs_layout_passes=False, use_tc_tiling_on_sc=False),
        scratch_types=[
            pltpu.VMEM((n_poses * 8,), F32),
            pltpu.VMEM((n_poses * 17,), F32),
            pltpu.VMEM((L,), F32),
            pltpu.VMEM((L,), F32),
        ] + 2 * [
            pltpu.VMEM((CHUNK // 128, 2, 128), F32),
            pltpu.VMEM((CHUNK,), F32),
            pltpu.VMEM((CHUNK // 128, 2, 128), F32),
            pltpu.VMEM((2 * CHUNK,), F32),
            pltpu.VMEM((CHUNK,), I32),
            pltpu.VMEM((CHUNK,), I32),
            pltpu.VMEM((2 * CHUNK,), F32),
        ] + [
            pltpu.SemaphoreType.DMA,
            pltpu.SemaphoreType.DMA,
            pltpu.SemaphoreType.DMA,
            pltpu.SemaphoreType.DMA,
        ],
    )
    def run(pose_hbm, rth_hbm, elev_hbm, base_hbm, w_hbm, sidx_hbm, tidx_hbm,
            sx_hbm, sy_hbm, out_hbm,
            pose_v, rt_v, sx_v, sy_v,
            rth_v0, elev_v0, base_v0, w_v0, sidx_v0, tidx_v0, out_v0,
            rth_v1, elev_v1, base_v1, w_v1, sidx_v1, tidx_v1, out_v1,
            sem_in0, sem_in1, sem_out0, sem_out1):
        wid = lax.axis_index("s") * NC + lax.axis_index("c")
        base0 = wid * pw
        nblk = CHUNK // 128
        pltpu.sync_copy(pose_hbm, pose_v)
        pltpu.sync_copy(sx_hbm, sx_v)
        pltpu.sync_copy(sy_hbm, sy_v)
        sxv = sx_v[...]
        syv = sy_v[...]
        iota = lax.iota(I32, L)
        iota2 = 2 * iota

        def build_rt(g, carry):
            pidx = g * L + iota
            a = lax.shift_left(pidx, 3)
            tx0 = plsc.load_gather(pose_v, [a])
            ty0 = plsc.load_gather(pose_v, [a + 1])
            tz0 = plsc.load_gather(pose_v, [a + 2])
            qx = plsc.load_gather(pose_v, [a + 3])
            qy = plsc.load_gather(pose_v, [a + 4])
            qz = plsc.load_gather(pose_v, [a + 5])
            qw = plsc.load_gather(pose_v, [a + 6])
            x2 = qx + qx
            y2 = qy + qy
            z2 = qz + qz
            xx2 = x2 * qx
            yy2 = y2 * qy
            zz2 = z2 * qz
            xy2 = x2 * qy
            xz2 = x2 * qz
            yz2 = y2 * qz
            wx2 = x2 * qw
            wy2 = y2 * qw
            wz2 = z2 * qw
            d = lax.shift_left(pidx, 4) + pidx
            plsc.store_scatter(rt_v, [d], 1.0 - (yy2 + zz2))
            plsc.store_scatter(rt_v, [d + 1], xy2 - wz2)
            plsc.store_scatter(rt_v, [d + 2], xz2 + wy2)
            plsc.store_scatter(rt_v, [d + 3], xy2 + wz2)
            plsc.store_scatter(rt_v, [d + 4], 1.0 - (xx2 + zz2))
            plsc.store_scatter(rt_v, [d + 5], yz2 - wx2)
            plsc.store_scatter(rt_v, [d + 6], xz2 - wy2)
            plsc.store_scatter(rt_v, [d + 7], yz2 + wx2)
            plsc.store_scatter(rt_v, [d + 8], 1.0 - (xx2 + yy2))
            plsc.store_scatter(rt_v, [d + 9], tx0)
            plsc.store_scatter(rt_v, [d + 10], ty0)
            plsc.store_scatter(rt_v, [d + 11], tz0)
            return carry

        lax.fori_loop(0, n_poses // L, build_rt, 0)

        slots = [
            (rth_v0, elev_v0, base_v0, w_v0, sidx_v0, tidx_v0, out_v0,
             sem_in0, sem_out0),
            (rth_v1, elev_v1, base_v1, w_v1, sidx_v1, tidx_v1, out_v1,
             sem_in1, sem_out1),
        ]

        def issue_in(c):
            rth_v, elev_v, base_v, w_v, sidx_v, tidx_v, _, sem, _ = slots[c % 2]
            base = base0 + c * CHUNK
            blk0 = base // 128
            return [
                pltpu.async_copy(rth_hbm.at[pl.ds(blk0, nblk)], rth_v, sem),
                pltpu.async_copy(elev_hbm.at[pl.ds(base, CHUNK)], elev_v, sem),
                pltpu.async_copy(base_hbm.at[pl.ds(blk0, nblk)], base_v, sem),
                pltpu.async_copy(w_hbm.at[pl.ds(2 * base, 2 * CHUNK)], w_v, sem),
                pltpu.async_copy(sidx_hbm.at[pl.ds(base, CHUNK)], sidx_v, sem),
                pltpu.async_copy(tidx_hbm.at[pl.ds(base, CHUNK)], tidx_v, sem),
            ]

        def compute_chunk(c):
            rth_v, elev_v, base_v, w_v, sidx_v, tidx_v, out_v, _, _ = slots[c % 2]

            def grp(j, carry2):
                off = j * L
                b = lax.shift_right_logical(j, 3)
                o = lax.shift_left(lax.bitwise_and(j, 7), 4)
                si = sidx_v[pl.ds(off, L)]
                ti = tidx_v[pl.ds(off, L)]
                sa = lax.shift_left(si, 4) + si
                ta = lax.shift_left(ti, 4) + ti
                sp = [plsc.load_gather(rt_v, [sa + k]) for k in range(12)]
                tp = [plsc.load_gather(rt_v, [ta + k]) for k in range(12)]
                r = rth_v[b, 0, pl.ds(o, L)]
                th = rth_v[b, 1, pl.ds(o, L)]
                ph = elev_v[pl.ds(off, L)]
                bx = base_v[b, 0, pl.ds(o, L)]
                by = base_v[b, 1, pl.ds(o, L)]
                i2 = 2 * off + iota2
                wx = plsc.load_gather(w_v, [i2])
                wy = plsc.load_gather(w_v, [i2 + 1])
                ex, ey = _edge_math(r, th, ph, sp, tp, bx, by, wx, wy, sxv, syv)
                plsc.store_scatter(out_v, [i2], ex)
                plsc.store_scatter(out_v, [i2 + 1], ey)
                return carry2

            lax.fori_loop(0, groups, grp, 0, unroll=2)

        def issue_out(c):
            out_v = slots[c % 2][6]
            sem = slots[c % 2][8]
            base = base0 + c * CHUNK
            return pltpu.async_copy(out_v, out_hbm.at[pl.ds(2 * base, 2 * CHUNK)],
                                    sem)

        pending_in = {0: issue_in(0)}
        pending_out = {}
        for c in range(n_chunks):
            for d in pending_in.pop(c):
                d.wait()
            if c + 1 < n_chunks:
                pending_in[c + 1] = issue_in(c + 1)
            if c - 2 in pending_out:
                pending_out.pop(c - 2).wait()
            compute_chunk(c)
            pending_out[c] = issue_out(c)
        for c in sorted(pending_out):
            pending_out.pop(c).wait()

    return run


def kernel(poses, elevation_angle_active, patch_coords_r_theta, coords_baseline,
           weights_1d, scale, source_frame_idx, target_frame_idx, patch_idx,
           inverse_patch_idx):
    n_edges = source_frame_idx.shape[0]
    n_poses = poses.shape[1]
    pose_pad = jnp.concatenate(
        [poses[0], jnp.zeros((n_poses, 1), F32)], axis=-1).reshape(-1)
    sx = jnp.broadcast_to(scale.reshape(2)[0], (L,))
    sy = jnp.broadcast_to(scale.reshape(2)[1], (L,))
    nb = n_edges // 128
    rth3 = patch_coords_r_theta.reshape(nb, 128, 2).transpose(0, 2, 1)
    base3 = coords_baseline.reshape(nb, 128, 2).transpose(0, 2, 1)
    elev1 = elevation_angle_active.reshape(n_edges)
    run = _sc_call(n_edges, n_poses)
    out = run(pose_pad, rth3, elev1, base3, weights_1d,
              source_frame_idx, target_frame_idx, sx, sy)
    return out.reshape(1, 2 * n_edges)
